# baseline (speedup 1.0000x reference)
import functools

import numpy as np
import jax
import jax.numpy as jnp
from jax.experimental import pallas as pl
from jax.experimental.pallas import tpu as pltpu

_EPS = 1e-06
_ROWS_PER_BLOCK = 8


def _two_sum(a, b):
    s = a + b
    bb = s - a
    err = (a - (s - bb)) + (b - bb)
    return s, err


def _quick_two_sum(s, e):
    h = s + e
    return h, e - (h - s)


def _split(a):
    t = a * 4097.0
    hi = t - (t - a)
    return hi, a - hi


def _two_prod(a, b):
    p = a * b
    ah, al = _split(a)
    bh, bl = _split(b)
    err = ((ah * bh - p) + ah * bl + al * bh) + al * bl
    return p, err


def _ds_add(ah, al, bh, bl):
    s, e = _two_sum(ah, bh)
    e = e + (al + bl)
    return _quick_two_sum(s, e)


def _ds_mul(ah, al, bh, bl):
    p, e = _two_prod(ah, bh)
    e = e + (ah * bl + al * bh)
    return _quick_two_sum(p, e)


def _exact_level_sums(x, log2_deltas):
    sums = []
    r = x
    last = len(log2_deltas) - 1
    for i, k in enumerate(log2_deltas):
        c = np.float32(1.5 * 2.0 ** (23 + k))
        h = (r + c) - c
        sums.append(jnp.sum(h, axis=(1, 2), keepdims=True))
        if i != last:
            r = r - h
    return sums


def _ds_sum_parts(parts):
    h, l = parts[0], jnp.zeros_like(parts[0])
    for p in parts[1:]:
        h, l = _ds_add(h, l, p, jnp.zeros_like(p))
    return h, l


def _decode_weights():
    p = np.arange(128)
    q = np.arange(128)
    same_seg = (p[:, None] // 32) == (q[None, :] // 32)
    k = p % 32
    w_hi = np.where(same_seg & (k < 16)[:, None], 2.0 ** (15 - k)[:, None], 0.0)
    w_lo = np.where(same_seg & (k >= 16)[:, None], 2.0 ** (31 - k)[:, None], 0.0)
    return w_hi.astype(jnp.bfloat16), w_lo.astype(jnp.bfloat16)


def _ln_kernel(x_ref, whi_ref, wlo_ref, o_ref, *, n):
    r_blk = x_ref.shape[0]
    rows_m = r_blk * x_ref.shape[1]
    bits = x_ref[...]
    b16 = bits.reshape(rows_m, 128).astype(jnp.bfloat16)

    hi = jnp.dot(b16, whi_ref[...], preferred_element_type=jnp.float32)
    lo = jnp.dot(b16, wlo_ref[...], preferred_element_type=jnp.float32)
    u = (hi.astype(jnp.uint32) << 16) | lo.astype(jnp.uint32)
    xf = jax.lax.bitcast_convert_type(u, jnp.float32)
    xf3 = xf.reshape(r_blk, x_ref.shape[1], 128)

    lane = jax.lax.broadcasted_iota(jnp.int32, (1, 1, 128), 2)

    rn_hi = np.float32(1.0 / n)
    rn_lo = np.float32(1.0 / n - float(rn_hi))

    srow = jax.lax.broadcasted_iota(jnp.int32, (1, x_ref.shape[1], 1), 1)
    rep_mask = (lane & 31) == (srow >> 3)
    zm = jnp.where(rep_mask, xf3, 0.0)
    zc = jnp.sum(zm.reshape(r_blk, x_ref.shape[1] // 8, 8, 128), axis=1)

    sh, sl = _ds_sum_parts(_exact_level_sums(zc, (-8, -20, -32, -44)))
    mh, ml = _ds_mul(sh, sl, rn_hi, rn_lo)

    zero = jnp.zeros_like(xf3)
    xch, xcl = _ds_add(xf3, zero, -mh, -ml)

    valid = (lane & 31) < (x_ref.shape[1] // 8)
    zch, zcl = _ds_add(zc, jnp.zeros_like(zc), -mh, -ml)
    sqh, sql = _ds_mul(zch, zcl, zch, zcl)
    parts = _exact_level_sums(jnp.where(valid, sqh, 0.0), (-5, -17, -29, -41))
    parts += _exact_level_sums(jnp.where(valid, sql, 0.0), (-29, -41))
    vh, vl = _ds_sum_parts(parts)
    vh, vl = _ds_mul(vh, vl, rn_hi, rn_lo)
    eps_hi = np.float32(_EPS)
    eps_lo = np.float32(_EPS - float(eps_hi))
    ah, al = _ds_add(vh, vl, eps_hi, eps_lo)

    yh = jax.lax.rsqrt(ah)
    yl = jnp.zeros_like(yh)
    for _ in range(3):
        t2h, t2l = _ds_mul(yh, yl, yh, yl)
        t3h, t3l = _ds_mul(ah, al, t2h, t2l)
        t4h, t4l = _ds_add(np.float32(3.0), np.float32(0.0), -t3h, -t3l)
        t5h, t5l = _ds_mul(yh, yl, t4h, t4l)
        yh, yl = t5h * 0.5, t5l * 0.5

    oh, _ = _ds_mul(xch, xcl, yh, yl)

    uo = jax.lax.bitcast_convert_type(oh, jnp.uint32)
    shift = (31 - (lane & 31)).astype(jnp.uint32)
    o_ref[...] = ((uo >> shift) & jnp.uint32(1)).astype(jnp.float32)


@jax.jit
def kernel(x):
    orig_shape = x.shape
    n = orig_shape[-2]
    rows = 1
    for d in orig_shape[:-2]:
        rows *= d
    seg_rows = n * 32 // 128
    xr = x.reshape(rows, seg_rows, 128)

    w_hi, w_lo = _decode_weights()
    r_blk = _ROWS_PER_BLOCK
    grid = (rows // r_blk,)
    out = pl.pallas_call(
        functools.partial(_ln_kernel, n=n),
        grid=grid,
        in_specs=[
            pl.BlockSpec((r_blk, seg_rows, 128),
                         lambda i: (i, jnp.int32(0), jnp.int32(0))),
            pl.BlockSpec((128, 128),
                         lambda i: (jnp.int32(0), jnp.int32(0))),
            pl.BlockSpec((128, 128),
                         lambda i: (jnp.int32(0), jnp.int32(0))),
        ],
        out_specs=pl.BlockSpec((r_blk, seg_rows, 128),
                               lambda i: (i, jnp.int32(0), jnp.int32(0))),
        out_shape=jax.ShapeDtypeStruct((rows, seg_rows, 128), jnp.float32),
        compiler_params=pltpu.CompilerParams(
            dimension_semantics=("parallel",),
            vmem_limit_bytes=100 * 1024 * 1024,
        ),
    )(xr, w_hi, w_lo)
    return out.reshape(orig_shape)


# speedup vs baseline: 1.0264x; 1.0264x over previous
import functools

import numpy as np
import jax
import jax.numpy as jnp
from jax.experimental import pallas as pl
from jax.experimental.pallas import tpu as pltpu

_EPS = 1e-06
_ROWS_PER_BLOCK = 32


def _two_sum(a, b):
    s = a + b
    bb = s - a
    err = (a - (s - bb)) + (b - bb)
    return s, err


def _quick_two_sum(s, e):
    h = s + e
    return h, e - (h - s)


def _split(a):
    t = a * 4097.0
    hi = t - (t - a)
    return hi, a - hi


def _two_prod(a, b):
    p = a * b
    ah, al = _split(a)
    bh, bl = _split(b)
    err = ((ah * bh - p) + ah * bl + al * bh) + al * bl
    return p, err


def _ds_add(ah, al, bh, bl):
    s, e = _two_sum(ah, bh)
    e = e + (al + bl)
    return _quick_two_sum(s, e)


def _ds_mul(ah, al, bh, bl):
    p, e = _two_prod(ah, bh)
    e = e + (ah * bl + al * bh)
    return _quick_two_sum(p, e)


def _exact_level_sums(x, log2_deltas):
    sums = []
    r = x
    last = len(log2_deltas) - 1
    for i, k in enumerate(log2_deltas):
        c = np.float32(1.5 * 2.0 ** (23 + k))
        h = (r + c) - c
        sums.append(jnp.sum(h, axis=(1, 2), keepdims=True))
        if i != last:
            r = r - h
    return sums


def _ds_sum_parts(parts):
    h, l = parts[0], jnp.zeros_like(parts[0])
    for p in parts[1:]:
        h, l = _ds_add(h, l, p, jnp.zeros_like(p))
    return h, l


def _decode_weights():
    p = np.arange(128)
    q = np.arange(128)
    same_seg = (p[:, None] // 32) == (q[None, :] // 32)
    k = p % 32
    w_hi = np.where(same_seg & (k < 16)[:, None], 2.0 ** (15 - k)[:, None], 0.0)
    w_lo = np.where(same_seg & (k >= 16)[:, None], 2.0 ** (31 - k)[:, None], 0.0)
    return w_hi.astype(jnp.bfloat16), w_lo.astype(jnp.bfloat16)


def _ln_kernel(x_ref, whi_ref, wlo_ref, o_ref, *, n):
    r_blk = x_ref.shape[0]
    rows_m = r_blk * x_ref.shape[1]
    bits = x_ref[...]
    b16 = bits.reshape(rows_m, 128).astype(jnp.bfloat16)

    hi = jnp.dot(b16, whi_ref[...], preferred_element_type=jnp.float32)
    lo = jnp.dot(b16, wlo_ref[...], preferred_element_type=jnp.float32)
    u = (hi.astype(jnp.uint32) << 16) | lo.astype(jnp.uint32)
    xf = jax.lax.bitcast_convert_type(u, jnp.float32)
    xf3 = xf.reshape(r_blk, x_ref.shape[1], 128)

    lane = jax.lax.broadcasted_iota(jnp.int32, (1, 1, 128), 2)

    rn_hi = np.float32(1.0 / n)
    rn_lo = np.float32(1.0 / n - float(rn_hi))

    srow = jax.lax.broadcasted_iota(jnp.int32, (1, x_ref.shape[1], 1), 1)
    rep_mask = (lane & 31) == (srow >> 3)
    zm = jnp.where(rep_mask, xf3, 0.0)
    zc = jnp.sum(zm.reshape(r_blk, x_ref.shape[1] // 8, 8, 128), axis=1)

    sh, sl = _ds_sum_parts(_exact_level_sums(zc, (-8, -20, -32, -44)))
    mh, ml = _ds_mul(sh, sl, rn_hi, rn_lo)

    zero = jnp.zeros_like(xf3)
    xch, xcl = _ds_add(xf3, zero, -mh, -ml)

    valid = (lane & 31) < (x_ref.shape[1] // 8)
    zch, zcl = _ds_add(zc, jnp.zeros_like(zc), -mh, -ml)
    sqh, sql = _ds_mul(zch, zcl, zch, zcl)
    parts = _exact_level_sums(jnp.where(valid, sqh, 0.0), (-5, -17, -29, -41))
    parts += _exact_level_sums(jnp.where(valid, sql, 0.0), (-29, -41))
    vh, vl = _ds_sum_parts(parts)
    vh, vl = _ds_mul(vh, vl, rn_hi, rn_lo)
    eps_hi = np.float32(_EPS)
    eps_lo = np.float32(_EPS - float(eps_hi))
    ah, al = _ds_add(vh, vl, eps_hi, eps_lo)

    yh = jax.lax.rsqrt(ah)
    yl = jnp.zeros_like(yh)
    for _ in range(3):
        t2h, t2l = _ds_mul(yh, yl, yh, yl)
        t3h, t3l = _ds_mul(ah, al, t2h, t2l)
        t4h, t4l = _ds_add(np.float32(3.0), np.float32(0.0), -t3h, -t3l)
        t5h, t5l = _ds_mul(yh, yl, t4h, t4l)
        yh, yl = t5h * 0.5, t5l * 0.5

    oh, _ = _ds_mul(xch, xcl, yh, yl)

    uo = jax.lax.bitcast_convert_type(oh, jnp.uint32)
    shift = (31 - (lane & 31)).astype(jnp.uint32)
    o_ref[...] = ((uo >> shift) & jnp.uint32(1)).astype(jnp.float32)


@jax.jit
def kernel(x):
    orig_shape = x.shape
    n = orig_shape[-2]
    rows = 1
    for d in orig_shape[:-2]:
        rows *= d
    seg_rows = n * 32 // 128
    xr = x.reshape(rows, seg_rows, 128)

    w_hi, w_lo = _decode_weights()
    r_blk = _ROWS_PER_BLOCK
    grid = (rows // r_blk,)
    out = pl.pallas_call(
        functools.partial(_ln_kernel, n=n),
        grid=grid,
        in_specs=[
            pl.BlockSpec((r_blk, seg_rows, 128),
                         lambda i: (i, jnp.int32(0), jnp.int32(0))),
            pl.BlockSpec((128, 128),
                         lambda i: (jnp.int32(0), jnp.int32(0))),
            pl.BlockSpec((128, 128),
                         lambda i: (jnp.int32(0), jnp.int32(0))),
        ],
        out_specs=pl.BlockSpec((r_blk, seg_rows, 128),
                               lambda i: (i, jnp.int32(0), jnp.int32(0))),
        out_shape=jax.ShapeDtypeStruct((rows, seg_rows, 128), jnp.float32),
        compiler_params=pltpu.CompilerParams(
            dimension_semantics=("parallel",),
            vmem_limit_bytes=100 * 1024 * 1024,
        ),
    )(xr, w_hi, w_lo)
    return out.reshape(orig_shape)


# speedup vs baseline: 1.6017x; 1.5605x over previous
import functools

import numpy as np
import jax
import jax.numpy as jnp
from jax.experimental import pallas as pl
from jax.experimental.pallas import tpu as pltpu

_EPS = 1e-06
_ROWS_PER_BLOCK = 16


def _two_sum(a, b):
    s = a + b
    bb = s - a
    err = (a - (s - bb)) + (b - bb)
    return s, err


def _quick_two_sum(s, e):
    h = s + e
    return h, e - (h - s)


def _split(a):
    t = a * 4097.0
    hi = t - (t - a)
    return hi, a - hi


def _two_prod(a, b):
    p = a * b
    ah, al = _split(a)
    bh, bl = _split(b)
    err = ((ah * bh - p) + ah * bl + al * bh) + al * bl
    return p, err


def _ds_add(ah, al, bh, bl):
    s, e = _two_sum(ah, bh)
    e = e + (al + bl)
    return _quick_two_sum(s, e)


def _ds_mul(ah, al, bh, bl):
    p, e = _two_prod(ah, bh)
    e = e + (ah * bl + al * bh)
    return _quick_two_sum(p, e)


def _exact_level_sums(x, log2_deltas):
    sums = []
    r = x
    last = len(log2_deltas) - 1
    for i, k in enumerate(log2_deltas):
        c = np.float32(1.5 * 2.0 ** (23 + k))
        h = (r + c) - c
        sums.append(jnp.sum(h, axis=-1, keepdims=True))
        if i != last:
            r = r - h
    return sums


def _ds_sum_parts(parts):
    h, l = parts[0], jnp.zeros_like(parts[0])
    for p in parts[1:]:
        h, l = _ds_add(h, l, p, jnp.zeros_like(p))
    return h, l


def _ln_kernel(x_ref, o_ref, hi_s, lo_s, *, n):
    bits = x_ref[...]

    k = jax.lax.broadcasted_iota(jnp.int32, (1, 1, 32), 2)
    pow2 = (jnp.int32(1) << (15 - (k & 15))).astype(jnp.float32)
    w_hi = jnp.where(k < 16, pow2, 0.0)
    w_lo = jnp.where(k >= 16, pow2, 0.0)
    hi_s[...] = jnp.sum(bits * w_hi, axis=-1)
    lo_s[...] = jnp.sum(bits * w_lo, axis=-1)
    hi_f = hi_s[...]
    lo_f = lo_s[...]
    u = (hi_f.astype(jnp.uint32) << 16) | lo_f.astype(jnp.uint32)
    xf = jax.lax.bitcast_convert_type(u, jnp.float32)

    rn_hi = np.float32(1.0 / n)
    rn_lo = np.float32(1.0 / n - float(rn_hi))

    sh, sl = _ds_sum_parts(_exact_level_sums(xf, (-8, -20, -32, -44)))
    mh, ml = _ds_mul(sh, sl, rn_hi, rn_lo)

    zero = jnp.zeros_like(xf)
    xch, xcl = _ds_add(xf, zero, -mh, -ml)

    sqh, sql = _ds_mul(xch, xcl, xch, xcl)
    parts = _exact_level_sums(sqh, (-5, -17, -29, -41))
    parts += _exact_level_sums(sql, (-29, -41))
    vh, vl = _ds_sum_parts(parts)
    vh, vl = _ds_mul(vh, vl, rn_hi, rn_lo)
    eps_hi = np.float32(_EPS)
    eps_lo = np.float32(_EPS - float(eps_hi))
    ah, al = _ds_add(vh, vl, eps_hi, eps_lo)

    yh = jax.lax.rsqrt(ah)
    yl = jnp.zeros_like(yh)
    for _ in range(3):
        t2h, t2l = _ds_mul(yh, yl, yh, yl)
        t3h, t3l = _ds_mul(ah, al, t2h, t2l)
        t4h, t4l = _ds_add(np.float32(3.0), np.float32(0.0), -t3h, -t3l)
        t5h, t5l = _ds_mul(yh, yl, t4h, t4l)
        yh, yl = t5h * 0.5, t5l * 0.5

    oh, _ = _ds_mul(xch, xcl, yh, yl)

    uo = jax.lax.bitcast_convert_type(oh, jnp.uint32)
    uo3 = jax.lax.broadcast_in_dim(uo, uo.shape + (32,), (0, 1))
    shift = (31 - k).astype(jnp.uint32)
    o_ref[...] = ((uo3 >> shift) & jnp.uint32(1)).astype(jnp.float32)


@jax.jit
def kernel(x):
    orig_shape = x.shape
    n = orig_shape[-2]
    rows = 1
    for d in orig_shape[:-2]:
        rows *= d
    xr = x.reshape(rows, n, 32)

    r_blk = _ROWS_PER_BLOCK
    grid = (rows // r_blk,)
    out = pl.pallas_call(
        functools.partial(_ln_kernel, n=n),
        grid=grid,
        in_specs=[pl.BlockSpec(
            (r_blk, n, 32), lambda i: (i, jnp.int32(0), jnp.int32(0)))],
        out_specs=pl.BlockSpec(
            (r_blk, n, 32), lambda i: (i, jnp.int32(0), jnp.int32(0))),
        out_shape=jax.ShapeDtypeStruct((rows, n, 32), jnp.float32),
        scratch_shapes=[
            pltpu.VMEM((r_blk, n), jnp.float32),
            pltpu.VMEM((r_blk, n), jnp.float32),
        ],
        compiler_params=pltpu.CompilerParams(
            dimension_semantics=("parallel",),
            vmem_limit_bytes=100 * 1024 * 1024,
        ),
    )(xr)
    return out.reshape(orig_shape)


# speedup vs baseline: 1.6431x; 1.0259x over previous
import functools

import numpy as np
import jax
import jax.numpy as jnp
from jax.experimental import pallas as pl
from jax.experimental.pallas import tpu as pltpu

_EPS = 1e-06
_ROWS_PER_BLOCK = 32


def _two_sum(a, b):
    s = a + b
    bb = s - a
    err = (a - (s - bb)) + (b - bb)
    return s, err


def _quick_two_sum(s, e):
    h = s + e
    return h, e - (h - s)


def _split(a):
    t = a * 4097.0
    hi = t - (t - a)
    return hi, a - hi


def _two_prod(a, b):
    p = a * b
    ah, al = _split(a)
    bh, bl = _split(b)
    err = ((ah * bh - p) + ah * bl + al * bh) + al * bl
    return p, err


def _ds_add(ah, al, bh, bl):
    s, e = _two_sum(ah, bh)
    e = e + (al + bl)
    return _quick_two_sum(s, e)


def _ds_mul(ah, al, bh, bl):
    p, e = _two_prod(ah, bh)
    e = e + (ah * bl + al * bh)
    return _quick_two_sum(p, e)


def _exact_level_sums(x, log2_deltas):
    sums = []
    r = x
    last = len(log2_deltas) - 1
    for i, k in enumerate(log2_deltas):
        c = np.float32(1.5 * 2.0 ** (23 + k))
        h = (r + c) - c
        sums.append(jnp.sum(h, axis=-1, keepdims=True))
        if i != last:
            r = r - h
    return sums


def _ds_sum_parts(parts):
    h, l = parts[0], jnp.zeros_like(parts[0])
    for p in parts[1:]:
        h, l = _ds_add(h, l, p, jnp.zeros_like(p))
    return h, l


def _ln_kernel(x_ref, o_ref, hi_s, lo_s, *, n):
    bits = x_ref[...]

    k = jax.lax.broadcasted_iota(jnp.int32, (1, 1, 32), 2)
    pow2 = (jnp.int32(1) << (15 - (k & 15))).astype(jnp.float32)
    w_hi = jnp.where(k < 16, pow2, 0.0)
    w_lo = jnp.where(k >= 16, pow2, 0.0)
    hi_s[...] = jnp.sum(bits * w_hi, axis=-1)
    lo_s[...] = jnp.sum(bits * w_lo, axis=-1)
    hi_f = hi_s[...]
    lo_f = lo_s[...]
    u = (hi_f.astype(jnp.uint32) << 16) | lo_f.astype(jnp.uint32)
    xf = jax.lax.bitcast_convert_type(u, jnp.float32)

    rn_hi = np.float32(1.0 / n)
    rn_lo = np.float32(1.0 / n - float(rn_hi))

    sh, sl = _ds_sum_parts(_exact_level_sums(xf, (-8, -20, -32, -44)))
    mh, ml = _ds_mul(sh, sl, rn_hi, rn_lo)

    zero = jnp.zeros_like(xf)
    xch, xcl = _ds_add(xf, zero, -mh, -ml)

    sqh, sql = _ds_mul(xch, xcl, xch, xcl)
    parts = _exact_level_sums(sqh, (-5, -17, -29, -41))
    parts += _exact_level_sums(sql, (-29, -41))
    vh, vl = _ds_sum_parts(parts)
    vh, vl = _ds_mul(vh, vl, rn_hi, rn_lo)
    eps_hi = np.float32(_EPS)
    eps_lo = np.float32(_EPS - float(eps_hi))
    ah, al = _ds_add(vh, vl, eps_hi, eps_lo)

    yh = jax.lax.rsqrt(ah)
    yl = jnp.zeros_like(yh)
    for _ in range(3):
        t2h, t2l = _ds_mul(yh, yl, yh, yl)
        t3h, t3l = _ds_mul(ah, al, t2h, t2l)
        t4h, t4l = _ds_add(np.float32(3.0), np.float32(0.0), -t3h, -t3l)
        t5h, t5l = _ds_mul(yh, yl, t4h, t4l)
        yh, yl = t5h * 0.5, t5l * 0.5

    oh, _ = _ds_mul(xch, xcl, yh, yl)

    uo = jax.lax.bitcast_convert_type(oh, jnp.uint32)
    uo3 = jax.lax.broadcast_in_dim(uo, uo.shape + (32,), (0, 1))
    shift = (31 - k).astype(jnp.uint32)
    o_ref[...] = ((uo3 >> shift) & jnp.uint32(1)).astype(jnp.float32)


@jax.jit
def kernel(x):
    orig_shape = x.shape
    n = orig_shape[-2]
    rows = 1
    for d in orig_shape[:-2]:
        rows *= d
    xr = x.reshape(rows, n, 32)

    r_blk = _ROWS_PER_BLOCK
    grid = (rows // r_blk,)
    out = pl.pallas_call(
        functools.partial(_ln_kernel, n=n),
        grid=grid,
        in_specs=[pl.BlockSpec(
            (r_blk, n, 32), lambda i: (i, jnp.int32(0), jnp.int32(0)))],
        out_specs=pl.BlockSpec(
            (r_blk, n, 32), lambda i: (i, jnp.int32(0), jnp.int32(0))),
        out_shape=jax.ShapeDtypeStruct((rows, n, 32), jnp.float32),
        scratch_shapes=[
            pltpu.VMEM((r_blk, n), jnp.float32),
            pltpu.VMEM((r_blk, n), jnp.float32),
        ],
        compiler_params=pltpu.CompilerParams(
            dimension_semantics=("parallel",),
            vmem_limit_bytes=100 * 1024 * 1024,
        ),
    )(xr)
    return out.reshape(orig_shape)
